# baseline (speedup 1.0000x reference)
import functools

import jax
import jax.numpy as jnp
from jax import lax
from jax.experimental import pallas as pl
from jax.experimental.pallas import tpu as pltpu
from jax.experimental.pallas import tpu_sc as plsc

VOCAB = 1000000
DIM = 32
B = 16384
L = 50

NUM_WORKERS = 32
B_PER_W = B // NUM_WORKERS
K_PER_W = B_PER_W * L
ROW_CHUNKS = B_PER_W // 16

V4 = VOCAB // 4
PROJ_BLK = 8192
PROJ_GRID = (V4 + PROJ_BLK - 1) // PROJ_BLK
V4_PAD = PROJ_GRID * PROJ_BLK


def _proj_body(w4_ref, tbl_ref, o_ref):
    blk = tbl_ref[...].reshape(PROJ_BLK, 128)
    o_ref[...] = jax.lax.dot_general(
        w4_ref[...], blk,
        dimension_numbers=(((1,), (1,)), ((), ())),
        precision=jax.lax.Precision.HIGHEST,
        preferred_element_type=jnp.float32,
    )


def _project_table(table_flat, w4t):
    return pl.pallas_call(
        _proj_body,
        grid=(PROJ_GRID,),
        in_specs=[
            pl.BlockSpec((4, 128), lambda i: (0, 0)),
            pl.BlockSpec((PROJ_BLK * 128,), lambda i: (i,)),
        ],
        out_specs=pl.BlockSpec((4, PROJ_BLK), lambda i: (0, i)),
        out_shape=jax.ShapeDtypeStruct((4, V4_PAD), jnp.float32),
    )(w4t, table_flat)


L_LO = L // 2
L_HI = L - L_LO
K_LO = L_LO * B_PER_W
K_HI = L_HI * B_PER_W


@functools.partial(
    pl.kernel,
    mesh=plsc.VectorSubcoreMesh(core_axis_name="c", subcore_axis_name="s"),
    out_type=jax.ShapeDtypeStruct((B,), jnp.float32),
    scratch_types=[
        pltpu.VMEM((K_LO,), jnp.int32),
        pltpu.VMEM((K_HI,), jnp.int32),
        pltpu.VMEM((K_PER_W,), jnp.float32),
        pltpu.VMEM((K_LO,), jnp.float32),
        pltpu.VMEM((K_HI,), jnp.float32),
        pltpu.VMEM((B_PER_W,), jnp.float32),
        pltpu.VMEM((16,), jnp.float32),
        pltpu.SemaphoreType.DMA,
        pltpu.SemaphoreType.DMA,
    ],
)
def _pool_kernel(idx_hbm, mask_hbm, t_hbm, bias_hbm, out_hbm,
                 idx0_v, idx1_v, m_v, g0_v, g1_v, out_v, b_v, sem0, sem1):
    n_cores = 2
    wid = lax.axis_index("s") * n_cores + lax.axis_index("c")
    row_base = wid * B_PER_W
    flat_base = wid * K_PER_W

    pltpu.sync_copy(idx_hbm.at[pl.ds(flat_base, K_LO)], idx0_v)
    cp0 = pltpu.async_copy(t_hbm.at[idx0_v], g0_v, sem0)
    pltpu.sync_copy(idx_hbm.at[pl.ds(flat_base + K_LO, K_HI)], idx1_v)
    cp1 = pltpu.async_copy(t_hbm.at[idx1_v], g1_v, sem1)
    pltpu.sync_copy(mask_hbm.at[pl.ds(flat_base, K_PER_W)], m_v)
    pltpu.sync_copy(bias_hbm, b_v)

    bias = b_v[...]
    accs = [bias] * ROW_CHUNKS

    def phase(g_ref, m_off, n_l, accs):
        def l_body(l, accs):
            base = l * B_PER_W
            return tuple(
                accs[c] + g_ref[pl.ds(base + c * 16, 16)]
                * m_v[pl.ds(m_off + base + c * 16, 16)]
                for c in range(ROW_CHUNKS)
            )
        return lax.fori_loop(0, n_l, l_body, tuple(accs))

    cp0.wait()
    accs = phase(g0_v, 0, L_LO, accs)
    cp1.wait()
    accs = phase(g1_v, K_LO, L_HI, accs)

    for c in range(ROW_CHUNKS):
        out_v[pl.ds(c * 16, 16)] = 1.0 / (1.0 + jnp.exp(-accs[c]))
    pltpu.sync_copy(out_v, out_hbm.at[pl.ds(row_base, B_PER_W)])


def kernel(lookup_tensor, mask, table, W, b):
    idx = lookup_tensor.astype(jnp.int32)
    idx = (idx % 4) * V4_PAD + idx // 4
    idx_flat = (idx.reshape(NUM_WORKERS, B_PER_W, L)
                .transpose(0, 2, 1)
                .reshape(B * L))
    mask_flat = (mask.astype(jnp.float32)
                 .reshape(NUM_WORKERS, B_PER_W, L)
                 .transpose(0, 2, 1)
                 .reshape(B * L))
    w32 = W.astype(jnp.float32).reshape(DIM)
    w4t = (jnp.eye(4, dtype=jnp.float32)[:, :, None] * w32[None, None, :]
           ).reshape(4, 128)
    bias_vec = jnp.broadcast_to(b.astype(jnp.float32).reshape(1), (16,))

    table_flat = table.astype(jnp.float32).reshape(VOCAB * DIM)
    t = _project_table(table_flat, w4t).reshape(4 * V4_PAD)
    out = _pool_kernel(idx_flat, mask_flat, t, bias_vec)
    return out.reshape(B, 1)


# speedup vs baseline: 3.3365x; 3.3365x over previous
import functools

import jax
import jax.numpy as jnp
from jax import lax
from jax.experimental import pallas as pl
from jax.experimental.pallas import tpu as pltpu
from jax.experimental.pallas import tpu_sc as plsc

VOCAB = 1000000
DIM = 32
B = 16384
L = 50

NUM_WORKERS = 32
B_PER_W = B // NUM_WORKERS
K_PER_W = B_PER_W * L
ROW_CHUNKS = B_PER_W // 16

TBLK = 8192
QC = TBLK // 4
PROJ_GRID = (VOCAB + TBLK - 1) // TBLK
VQ_PAD = PROJ_GRID * QC


def _proj_body(w4_ref, tbl_ref, o_ref):
    blk = tbl_ref[...]
    stacked = jnp.concatenate(
        [blk[:, q * QC:(q + 1) * QC] for q in range(4)], axis=0)
    o_ref[...] = jax.lax.dot_general(
        w4_ref[...], stacked,
        dimension_numbers=(((1,), (0,)), ((), ())),
        precision=jax.lax.Precision.HIGHEST,
        preferred_element_type=jnp.float32,
    )


def _project_table(table_t, w4t):
    return pl.pallas_call(
        _proj_body,
        grid=(PROJ_GRID,),
        in_specs=[
            pl.BlockSpec((4, 128), lambda i: (0, 0)),
            pl.BlockSpec((DIM, TBLK), lambda i: (0, i)),
        ],
        out_specs=pl.BlockSpec((4, QC), lambda i: (0, i)),
        out_shape=jax.ShapeDtypeStruct((4, VQ_PAD), jnp.float32),
    )(w4t, table_t)


L_LO = L // 2
L_HI = L - L_LO
K_LO = L_LO * B_PER_W
K_HI = L_HI * B_PER_W


@functools.partial(
    pl.kernel,
    mesh=plsc.VectorSubcoreMesh(core_axis_name="c", subcore_axis_name="s"),
    out_type=jax.ShapeDtypeStruct((B,), jnp.float32),
    scratch_types=[
        pltpu.VMEM((K_LO,), jnp.int32),
        pltpu.VMEM((K_HI,), jnp.int32),
        pltpu.VMEM((K_PER_W,), jnp.float32),
        pltpu.VMEM((K_LO,), jnp.float32),
        pltpu.VMEM((K_HI,), jnp.float32),
        pltpu.VMEM((B_PER_W,), jnp.float32),
        pltpu.VMEM((16,), jnp.float32),
        pltpu.SemaphoreType.DMA,
        pltpu.SemaphoreType.DMA,
    ],
)
def _pool_kernel(idx_hbm, mask_hbm, t_hbm, bias_hbm, out_hbm,
                 idx0_v, idx1_v, m_v, g0_v, g1_v, out_v, b_v, sem0, sem1):
    n_cores = 2
    wid = lax.axis_index("s") * n_cores + lax.axis_index("c")
    row_base = wid * B_PER_W
    flat_base = wid * K_PER_W

    pltpu.sync_copy(idx_hbm.at[pl.ds(flat_base, K_LO)], idx0_v)
    cp0 = pltpu.async_copy(t_hbm.at[idx0_v], g0_v, sem0)
    pltpu.sync_copy(idx_hbm.at[pl.ds(flat_base + K_LO, K_HI)], idx1_v)
    cp1 = pltpu.async_copy(t_hbm.at[idx1_v], g1_v, sem1)
    pltpu.sync_copy(mask_hbm.at[pl.ds(flat_base, K_PER_W)], m_v)
    pltpu.sync_copy(bias_hbm, b_v)

    bias = b_v[...]
    accs = [bias] * ROW_CHUNKS

    def phase(g_ref, m_off, n_l, accs):
        def l_body(l, accs):
            base = l * B_PER_W
            return tuple(
                accs[c] + g_ref[pl.ds(base + c * 16, 16)]
                * m_v[pl.ds(m_off + base + c * 16, 16)]
                for c in range(ROW_CHUNKS)
            )
        return lax.fori_loop(0, n_l, l_body, tuple(accs))

    cp0.wait()
    accs = phase(g0_v, 0, L_LO, accs)
    cp1.wait()
    accs = phase(g1_v, K_LO, L_HI, accs)

    for c in range(ROW_CHUNKS):
        out_v[pl.ds(c * 16, 16)] = 1.0 / (1.0 + jnp.exp(-accs[c]))
    pltpu.sync_copy(out_v, out_hbm.at[pl.ds(row_base, B_PER_W)])


def kernel(lookup_tensor, mask, table, W, b):
    idx = lookup_tensor.astype(jnp.int32)
    idx = ((idx % TBLK) // QC) * VQ_PAD + (idx // TBLK) * QC + idx % QC
    idx_flat = (idx.reshape(NUM_WORKERS, B_PER_W, L)
                .transpose(0, 2, 1)
                .reshape(B * L))
    mask_flat = (mask.astype(jnp.float32)
                 .reshape(NUM_WORKERS, B_PER_W, L)
                 .transpose(0, 2, 1)
                 .reshape(B * L))
    w32 = W.astype(jnp.float32).reshape(DIM)
    w4t = (jnp.eye(4, dtype=jnp.float32)[:, :, None] * w32[None, None, :]
           ).reshape(4, 128)
    bias_vec = jnp.broadcast_to(b.astype(jnp.float32).reshape(1), (16,))

    table_t = table.astype(jnp.float32).T
    t = _project_table(table_t, w4t).reshape(4 * VQ_PAD)
    out = _pool_kernel(idx_flat, mask_flat, t, bias_vec)
    return out.reshape(B, 1)


# speedup vs baseline: 4.4851x; 1.3443x over previous
import functools

import jax
import jax.numpy as jnp
from jax import lax
from jax.experimental import pallas as pl
from jax.experimental.pallas import tpu as pltpu
from jax.experimental.pallas import tpu_sc as plsc

VOCAB = 1000000
DIM = 32
B = 16384
L = 50

NUM_WORKERS = 32
B_PER_W = B // NUM_WORKERS
K_PER_W = B_PER_W * L
ROW_CHUNKS = B_PER_W // 16

TBLK = 32768
QC = TBLK // 4
PROJ_GRID = (VOCAB + TBLK - 1) // TBLK
VQ_PAD = PROJ_GRID * QC


def _proj_body(w4_ref, tbl_ref, o_ref):
    blk = tbl_ref[...]
    stacked = jnp.concatenate(
        [blk[:, q * QC:(q + 1) * QC] for q in range(4)], axis=0)
    o_ref[...] = jax.lax.dot_general(
        w4_ref[...], stacked,
        dimension_numbers=(((1,), (0,)), ((), ())),
        precision=jax.lax.Precision.HIGHEST,
        preferred_element_type=jnp.float32,
    )


def _project_table(table_t, w4t):
    return pl.pallas_call(
        _proj_body,
        grid=(PROJ_GRID,),
        in_specs=[
            pl.BlockSpec((4, 128), lambda i: (0, 0)),
            pl.BlockSpec((DIM, TBLK), lambda i: (0, i)),
        ],
        out_specs=pl.BlockSpec((4, QC), lambda i: (0, i)),
        out_shape=jax.ShapeDtypeStruct((4, VQ_PAD), jnp.float32),
    )(w4t, table_t)


L_LO = L // 2
L_HI = L - L_LO
K_LO = L_LO * B_PER_W
K_HI = L_HI * B_PER_W


@functools.partial(
    pl.kernel,
    mesh=plsc.VectorSubcoreMesh(core_axis_name="c", subcore_axis_name="s"),
    out_type=jax.ShapeDtypeStruct((B,), jnp.float32),
    scratch_types=[
        pltpu.VMEM((K_LO,), jnp.int32),
        pltpu.VMEM((K_HI,), jnp.int32),
        pltpu.VMEM((K_PER_W,), jnp.float32),
        pltpu.VMEM((K_LO,), jnp.float32),
        pltpu.VMEM((K_HI,), jnp.float32),
        pltpu.VMEM((B_PER_W,), jnp.float32),
        pltpu.VMEM((16,), jnp.float32),
        pltpu.SemaphoreType.DMA,
        pltpu.SemaphoreType.DMA,
    ],
)
def _pool_kernel(idx_hbm, mask_hbm, t_hbm, bias_hbm, out_hbm,
                 idx0_v, idx1_v, m_v, g0_v, g1_v, out_v, b_v, sem0, sem1):
    n_cores = 2
    wid = lax.axis_index("s") * n_cores + lax.axis_index("c")
    row_base = wid * B_PER_W
    flat_base = wid * K_PER_W

    pltpu.sync_copy(idx_hbm.at[pl.ds(flat_base, K_LO)], idx0_v)
    cp0 = pltpu.async_copy(t_hbm.at[idx0_v], g0_v, sem0)
    pltpu.sync_copy(idx_hbm.at[pl.ds(flat_base + K_LO, K_HI)], idx1_v)
    cp1 = pltpu.async_copy(t_hbm.at[idx1_v], g1_v, sem1)
    pltpu.sync_copy(mask_hbm.at[pl.ds(flat_base, K_PER_W)], m_v)
    pltpu.sync_copy(bias_hbm, b_v)

    bias = b_v[...]
    accs = [bias] * ROW_CHUNKS

    def phase(g_ref, m_off, n_l, accs):
        def l_body(l, accs):
            base = l * B_PER_W
            return tuple(
                accs[c] + g_ref[pl.ds(base + c * 16, 16)]
                * m_v[pl.ds(m_off + base + c * 16, 16)]
                for c in range(ROW_CHUNKS)
            )
        return lax.fori_loop(0, n_l, l_body, tuple(accs))

    cp0.wait()
    accs = phase(g0_v, 0, L_LO, accs)
    cp1.wait()
    accs = phase(g1_v, K_LO, L_HI, accs)

    for c in range(ROW_CHUNKS):
        out_v[pl.ds(c * 16, 16)] = 1.0 / (1.0 + jnp.exp(-accs[c]))
    pltpu.sync_copy(out_v, out_hbm.at[pl.ds(row_base, B_PER_W)])


def kernel(lookup_tensor, mask, table, W, b):
    idx = lookup_tensor.astype(jnp.int32)
    idx = ((idx % TBLK) // QC) * VQ_PAD + (idx // TBLK) * QC + idx % QC
    idx_flat = (idx.reshape(NUM_WORKERS, B_PER_W, L)
                .transpose(0, 2, 1)
                .reshape(B * L))
    mask_flat = (mask.astype(jnp.float32)
                 .reshape(NUM_WORKERS, B_PER_W, L)
                 .transpose(0, 2, 1)
                 .reshape(B * L))
    w32 = W.astype(jnp.float32).reshape(DIM)
    w4t = (jnp.eye(4, dtype=jnp.float32)[:, :, None] * w32[None, None, :]
           ).reshape(4, 128)
    bias_vec = jnp.broadcast_to(b.astype(jnp.float32).reshape(1), (16,))

    table_t = table.astype(jnp.float32).T
    t = _project_table(table_t, w4t).reshape(4 * VQ_PAD)
    out = _pool_kernel(idx_flat, mask_flat, t, bias_vec)
    return out.reshape(B, 1)


# speedup vs baseline: 4.6019x; 1.0260x over previous
import functools

import jax
import jax.numpy as jnp
from jax import lax
from jax.experimental import pallas as pl
from jax.experimental.pallas import tpu as pltpu
from jax.experimental.pallas import tpu_sc as plsc

VOCAB = 1000000
DIM = 32
B = 16384
L = 50

NUM_WORKERS = 32
B_PER_W = B // NUM_WORKERS
K_PER_W = B_PER_W * L
ROW_CHUNKS = B_PER_W // 16

TBLK = 65536
QC = TBLK // 4
PROJ_GRID = (VOCAB + TBLK - 1) // TBLK
VQ_PAD = PROJ_GRID * QC


def _proj_body(w4_ref, tbl_ref, o_ref):
    blk = tbl_ref[...]
    stacked = jnp.concatenate(
        [blk[:, q * QC:(q + 1) * QC] for q in range(4)], axis=0)
    o_ref[...] = jax.lax.dot_general(
        w4_ref[...], stacked,
        dimension_numbers=(((1,), (0,)), ((), ())),
        precision=jax.lax.Precision.HIGHEST,
        preferred_element_type=jnp.float32,
    )


def _project_table(table_t, w4t):
    return pl.pallas_call(
        _proj_body,
        grid=(PROJ_GRID,),
        in_specs=[
            pl.BlockSpec((4, 128), lambda i: (0, 0)),
            pl.BlockSpec((DIM, TBLK), lambda i: (0, i)),
        ],
        out_specs=pl.BlockSpec((4, QC), lambda i: (0, i)),
        out_shape=jax.ShapeDtypeStruct((4, VQ_PAD), jnp.float32),
    )(w4t, table_t)


L_LO = L // 2
L_HI = L - L_LO
K_LO = L_LO * B_PER_W
K_HI = L_HI * B_PER_W


@functools.partial(
    pl.kernel,
    mesh=plsc.VectorSubcoreMesh(core_axis_name="c", subcore_axis_name="s"),
    out_type=jax.ShapeDtypeStruct((B,), jnp.float32),
    scratch_types=[
        pltpu.VMEM((K_LO,), jnp.int32),
        pltpu.VMEM((K_HI,), jnp.int32),
        pltpu.VMEM((K_PER_W,), jnp.float32),
        pltpu.VMEM((K_LO,), jnp.float32),
        pltpu.VMEM((K_HI,), jnp.float32),
        pltpu.VMEM((B_PER_W,), jnp.float32),
        pltpu.VMEM((16,), jnp.float32),
        pltpu.SemaphoreType.DMA,
        pltpu.SemaphoreType.DMA,
    ],
)
def _pool_kernel(idx_hbm, mask_hbm, t_hbm, bias_hbm, out_hbm,
                 idx0_v, idx1_v, m_v, g0_v, g1_v, out_v, b_v, sem0, sem1):
    n_cores = 2
    wid = lax.axis_index("s") * n_cores + lax.axis_index("c")
    row_base = wid * B_PER_W
    flat_base = wid * K_PER_W

    pltpu.sync_copy(idx_hbm.at[pl.ds(flat_base, K_LO)], idx0_v)
    cp0 = pltpu.async_copy(t_hbm.at[idx0_v], g0_v, sem0)
    pltpu.sync_copy(idx_hbm.at[pl.ds(flat_base + K_LO, K_HI)], idx1_v)
    cp1 = pltpu.async_copy(t_hbm.at[idx1_v], g1_v, sem1)
    pltpu.sync_copy(mask_hbm.at[pl.ds(flat_base, K_PER_W)], m_v)
    pltpu.sync_copy(bias_hbm, b_v)

    bias = b_v[...]
    accs = [bias] * ROW_CHUNKS

    def phase(g_ref, m_off, n_l, accs):
        def l_body(l, accs):
            base = l * B_PER_W
            return tuple(
                accs[c] + g_ref[pl.ds(base + c * 16, 16)]
                * m_v[pl.ds(m_off + base + c * 16, 16)]
                for c in range(ROW_CHUNKS)
            )
        return lax.fori_loop(0, n_l, l_body, tuple(accs))

    cp0.wait()
    accs = phase(g0_v, 0, L_LO, accs)
    cp1.wait()
    accs = phase(g1_v, K_LO, L_HI, accs)

    for c in range(ROW_CHUNKS):
        out_v[pl.ds(c * 16, 16)] = 1.0 / (1.0 + jnp.exp(-accs[c]))
    pltpu.sync_copy(out_v, out_hbm.at[pl.ds(row_base, B_PER_W)])


def kernel(lookup_tensor, mask, table, W, b):
    idx = lookup_tensor.astype(jnp.int32)
    idx = ((idx % TBLK) // QC) * VQ_PAD + (idx // TBLK) * QC + idx % QC
    idx_flat = (idx.reshape(NUM_WORKERS, B_PER_W, L)
                .transpose(0, 2, 1)
                .reshape(B * L))
    mask_flat = (mask.astype(jnp.float32)
                 .reshape(NUM_WORKERS, B_PER_W, L)
                 .transpose(0, 2, 1)
                 .reshape(B * L))
    w32 = W.astype(jnp.float32).reshape(DIM)
    w4t = (jnp.eye(4, dtype=jnp.float32)[:, :, None] * w32[None, None, :]
           ).reshape(4, 128)
    bias_vec = jnp.broadcast_to(b.astype(jnp.float32).reshape(1), (16,))

    table_t = table.astype(jnp.float32).T
    t = _project_table(table_t, w4t).reshape(4 * VQ_PAD)
    out = _pool_kernel(idx_flat, mask_flat, t, bias_vec)
    return out.reshape(B, 1)


# speedup vs baseline: 4.9164x; 1.0684x over previous
import functools

import jax
import jax.numpy as jnp
from jax import lax
from jax.experimental import pallas as pl
from jax.experimental.pallas import tpu as pltpu
from jax.experimental.pallas import tpu_sc as plsc

VOCAB = 1000000
DIM = 32
B = 16384
L = 50

NUM_WORKERS = 32
B_PER_W = B // NUM_WORKERS
K_PER_W = B_PER_W * L
ROW_CHUNKS = B_PER_W // 16

TBLK = 65536
QC = TBLK // 4
PROJ_GRID = (VOCAB + TBLK - 1) // TBLK


def _proj_body(w4_ref, tbl_ref, o_ref):
    blk = tbl_ref[...]
    stacked = jnp.concatenate(
        [blk[:, q * QC:(q + 1) * QC] for q in range(4)], axis=0)
    prod = jax.lax.dot_general(
        w4_ref[...], stacked,
        dimension_numbers=(((1,), (0,)), ((), ())),
        precision=jax.lax.Precision.HIGHEST,
        preferred_element_type=jnp.float32,
    )
    o_ref[...] = prod.reshape(TBLK)


def _project_table(table_t, w4t):
    return pl.pallas_call(
        _proj_body,
        grid=(PROJ_GRID,),
        in_specs=[
            pl.BlockSpec((4, 128), lambda i: (0, 0)),
            pl.BlockSpec((DIM, TBLK), lambda i: (0, i)),
        ],
        out_specs=pl.BlockSpec((TBLK,), lambda i: (i,)),
        out_shape=jax.ShapeDtypeStruct((PROJ_GRID * TBLK,), jnp.float32),
    )(w4t, table_t)


L_LO = L // 2
L_HI = L - L_LO
K_LO = L_LO * B_PER_W
K_HI = L_HI * B_PER_W


@functools.partial(
    pl.kernel,
    mesh=plsc.VectorSubcoreMesh(core_axis_name="c", subcore_axis_name="s"),
    out_type=jax.ShapeDtypeStruct((B,), jnp.float32),
    scratch_types=[
        pltpu.VMEM((K_LO,), jnp.int32),
        pltpu.VMEM((K_HI,), jnp.int32),
        pltpu.VMEM((K_PER_W,), jnp.float32),
        pltpu.VMEM((K_LO,), jnp.float32),
        pltpu.VMEM((K_HI,), jnp.float32),
        pltpu.VMEM((B_PER_W,), jnp.float32),
        pltpu.VMEM((16,), jnp.float32),
        pltpu.SemaphoreType.DMA,
        pltpu.SemaphoreType.DMA,
    ],
)
def _pool_kernel(idx_hbm, mask_hbm, t_hbm, bias_hbm, out_hbm,
                 idx0_v, idx1_v, m_v, g0_v, g1_v, out_v, b_v, sem0, sem1):
    n_cores = 2
    wid = lax.axis_index("s") * n_cores + lax.axis_index("c")
    row_base = wid * B_PER_W
    flat_base = wid * K_PER_W

    pltpu.sync_copy(idx_hbm.at[pl.ds(flat_base, K_LO)], idx0_v)
    cp0 = pltpu.async_copy(t_hbm.at[idx0_v], g0_v, sem0)
    pltpu.sync_copy(idx_hbm.at[pl.ds(flat_base + K_LO, K_HI)], idx1_v)
    cp1 = pltpu.async_copy(t_hbm.at[idx1_v], g1_v, sem1)
    pltpu.sync_copy(mask_hbm.at[pl.ds(flat_base, K_PER_W)], m_v)
    pltpu.sync_copy(bias_hbm, b_v)

    bias = b_v[...]
    accs = [bias] * ROW_CHUNKS

    def phase(g_ref, m_off, n_l, accs):
        def l_body(l, accs):
            base = l * B_PER_W
            return tuple(
                accs[c] + g_ref[pl.ds(base + c * 16, 16)]
                * m_v[pl.ds(m_off + base + c * 16, 16)]
                for c in range(ROW_CHUNKS)
            )
        return lax.fori_loop(0, n_l, l_body, tuple(accs))

    cp0.wait()
    accs = phase(g0_v, 0, L_LO, accs)
    cp1.wait()
    accs = phase(g1_v, K_LO, L_HI, accs)

    for c in range(ROW_CHUNKS):
        out_v[pl.ds(c * 16, 16)] = 1.0 / (1.0 + jnp.exp(-accs[c]))
    pltpu.sync_copy(out_v, out_hbm.at[pl.ds(row_base, B_PER_W)])


def kernel(lookup_tensor, mask, table, W, b):
    idx = lookup_tensor.astype(jnp.int32)
    idx_flat = (idx.reshape(NUM_WORKERS, B_PER_W, L)
                .transpose(0, 2, 1)
                .reshape(B * L))
    mask_flat = (mask.astype(jnp.float32)
                 .reshape(NUM_WORKERS, B_PER_W, L)
                 .transpose(0, 2, 1)
                 .reshape(B * L))
    w32 = W.astype(jnp.float32).reshape(DIM)
    w4t = (jnp.eye(4, dtype=jnp.float32)[:, :, None] * w32[None, None, :]
           ).reshape(4, 128)
    bias_vec = jnp.broadcast_to(b.astype(jnp.float32).reshape(1), (16,))

    table_t = table.astype(jnp.float32).T
    t = _project_table(table_t, w4t)
    out = _pool_kernel(idx_flat, mask_flat, t, bias_vec)
    return out.reshape(B, 1)


# speedup vs baseline: 4.9170x; 1.0001x over previous
import functools

import jax
import jax.numpy as jnp
from jax import lax
from jax.experimental import pallas as pl
from jax.experimental.pallas import tpu as pltpu
from jax.experimental.pallas import tpu_sc as plsc

VOCAB = 1000000
DIM = 32
B = 16384
L = 50

NUM_WORKERS = 32
B_PER_W = B // NUM_WORKERS
K_PER_W = B_PER_W * L
ROW_CHUNKS = B_PER_W // 16

TBLK = 131072
QC = TBLK // 4
PROJ_GRID = (VOCAB + TBLK - 1) // TBLK


def _proj_body(w4_ref, tbl_ref, o_ref):
    blk = tbl_ref[...]
    stacked = jnp.concatenate(
        [blk[:, q * QC:(q + 1) * QC] for q in range(4)], axis=0)
    prod = jax.lax.dot_general(
        w4_ref[...], stacked,
        dimension_numbers=(((1,), (0,)), ((), ())),
        precision=jax.lax.Precision.HIGHEST,
        preferred_element_type=jnp.float32,
    )
    o_ref[...] = prod.reshape(TBLK)


def _project_table(table_t, w4t):
    return pl.pallas_call(
        _proj_body,
        grid=(PROJ_GRID,),
        in_specs=[
            pl.BlockSpec((4, 128), lambda i: (0, 0)),
            pl.BlockSpec((DIM, TBLK), lambda i: (0, i)),
        ],
        out_specs=pl.BlockSpec((TBLK,), lambda i: (i,)),
        out_shape=jax.ShapeDtypeStruct((PROJ_GRID * TBLK,), jnp.float32),
    )(w4t, table_t)


L_LO = L // 2
L_HI = L - L_LO
K_LO = L_LO * B_PER_W
K_HI = L_HI * B_PER_W


@functools.partial(
    pl.kernel,
    mesh=plsc.VectorSubcoreMesh(core_axis_name="c", subcore_axis_name="s"),
    out_type=jax.ShapeDtypeStruct((B,), jnp.float32),
    scratch_types=[
        pltpu.VMEM((K_LO,), jnp.int32),
        pltpu.VMEM((K_HI,), jnp.int32),
        pltpu.VMEM((K_PER_W,), jnp.float32),
        pltpu.VMEM((K_LO,), jnp.float32),
        pltpu.VMEM((K_HI,), jnp.float32),
        pltpu.VMEM((B_PER_W,), jnp.float32),
        pltpu.VMEM((16,), jnp.float32),
        pltpu.SemaphoreType.DMA,
        pltpu.SemaphoreType.DMA,
    ],
)
def _pool_kernel(idx_hbm, mask_hbm, t_hbm, bias_hbm, out_hbm,
                 idx0_v, idx1_v, m_v, g0_v, g1_v, out_v, b_v, sem0, sem1):
    n_cores = 2
    wid = lax.axis_index("s") * n_cores + lax.axis_index("c")
    row_base = wid * B_PER_W
    flat_base = wid * K_PER_W

    pltpu.sync_copy(idx_hbm.at[pl.ds(flat_base, K_LO)], idx0_v)
    cp0 = pltpu.async_copy(t_hbm.at[idx0_v], g0_v, sem0)
    pltpu.sync_copy(idx_hbm.at[pl.ds(flat_base + K_LO, K_HI)], idx1_v)
    cp1 = pltpu.async_copy(t_hbm.at[idx1_v], g1_v, sem1)
    pltpu.sync_copy(mask_hbm.at[pl.ds(flat_base, K_PER_W)], m_v)
    pltpu.sync_copy(bias_hbm, b_v)

    bias = b_v[...]
    accs = [bias] * ROW_CHUNKS

    def phase(g_ref, m_off, n_l, accs):
        def l_body(l, accs):
            base = l * B_PER_W
            return tuple(
                accs[c] + g_ref[pl.ds(base + c * 16, 16)]
                * m_v[pl.ds(m_off + base + c * 16, 16)]
                for c in range(ROW_CHUNKS)
            )
        return lax.fori_loop(0, n_l, l_body, tuple(accs))

    cp0.wait()
    accs = phase(g0_v, 0, L_LO, accs)
    cp1.wait()
    accs = phase(g1_v, K_LO, L_HI, accs)

    for c in range(ROW_CHUNKS):
        out_v[pl.ds(c * 16, 16)] = 1.0 / (1.0 + jnp.exp(-accs[c]))
    pltpu.sync_copy(out_v, out_hbm.at[pl.ds(row_base, B_PER_W)])


def kernel(lookup_tensor, mask, table, W, b):
    idx = lookup_tensor.astype(jnp.int32)
    idx_flat = (idx.reshape(NUM_WORKERS, B_PER_W, L)
                .transpose(0, 2, 1)
                .reshape(B * L))
    mask_flat = (mask.astype(jnp.float32)
                 .reshape(NUM_WORKERS, B_PER_W, L)
                 .transpose(0, 2, 1)
                 .reshape(B * L))
    w32 = W.astype(jnp.float32).reshape(DIM)
    w4t = (jnp.eye(4, dtype=jnp.float32)[:, :, None] * w32[None, None, :]
           ).reshape(4, 128)
    bias_vec = jnp.broadcast_to(b.astype(jnp.float32).reshape(1), (16,))

    table_t = table.astype(jnp.float32).T
    t = _project_table(table_t, w4t)
    out = _pool_kernel(idx_flat, mask_flat, t, bias_vec)
    return out.reshape(B, 1)
